# baseline (speedup 1.0000x reference)
import jax
import jax.numpy as jnp
from jax.experimental import pallas as pl

_TOPK = 8
_LAMBD = 0.001
_WC = 256


def kernel(x, y, edge_index, neighbor_index, W1, b1, bn1_g, bn1_b, a1,
           W2, b2, bn2_g, bn2_b, a2, Wp1, bp1, bnp_g, bnp_b, ap, Wp2, bp2):
    N, D = x.shape
    src, dst = edge_index[0], edge_index[1]
    E = src.shape[0]

    deg = jnp.zeros((N,), jnp.float32).at[dst].add(1.0) + 1.0
    dinv = jax.lax.rsqrt(deg)

    def layer(xin, W, b, g, beta, a):
        h = xin @ W
        gsc = h * dinv[:, None]
        msg = jnp.zeros((N, D), jnp.float32).at[dst].add(gsc[src])
        out = dinv[:, None] * msg + (dinv * dinv)[:, None] * h + b
        m = jnp.mean(out, axis=0)
        v = jnp.var(out, axis=0)
        out = (out - m) / jnp.sqrt(v + 1e-5) * g + beta
        return jnp.where(out >= 0, out, a * out)

    x1 = layer(x, W1, b1, bn1_g, bn1_b, a1)
    student = layer(x1, W2, b2, bn2_g, bn2_b, a2)

    h = student @ Wp1 + bp1
    m = jnp.mean(h, axis=0)
    v = jnp.var(h, axis=0)
    h = (h - m) / jnp.sqrt(v + 1e-5) * bnp_g + bnp_b
    h = jnp.where(h >= 0, h, ap * h)
    pred = h @ Wp2 + bp2

    sn = student / jnp.clip(jnp.linalg.norm(student, axis=-1, keepdims=True), 1e-12)
    p = pred / jnp.clip(jnp.linalg.norm(pred, axis=-1, keepdims=True), 1e-12)
    t = sn

    ns, nd = neighbor_index[0], neighbor_index[1]
    key_s = jnp.sort(ns * N + nd)
    ns_s = key_s // N
    nd_s = key_s % N
    sim = jnp.sum(sn[ns_s] * t[nd_s], axis=-1)

    rdeg = jnp.zeros((N,), jnp.int32).at[ns].add(1)
    row_start = jnp.concatenate([jnp.zeros((1,), jnp.int32),
                                 jnp.cumsum(rdeg).astype(jnp.int32)])

    w = jnp.arange(_WC, dtype=jnp.int32)
    flat = row_start[:N, None] + w[None, :]
    validw = w[None, :] < rdeg[:, None]
    flat_c = jnp.minimum(flat, E - 1)
    T = jnp.where(validw, sim[flat_c], 0.0)
    C = jnp.where(validw, nd_s[flat_c], -1)

    S = T
    d_ = 1
    while d_ < _WC:
        C_sh = jnp.concatenate(
            [C[:, d_:], jnp.full((N, d_), -2, C.dtype)], axis=1)
        S_sh = jnp.concatenate(
            [S[:, d_:], jnp.zeros((N, d_), S.dtype)], axis=1)
        S = S + jnp.where(C == C_sh, S_sh, 0.0)
        d_ *= 2

    rep = jnp.concatenate(
        [jnp.ones((N, 1), bool), C[:, 1:] != C[:, :-1]], axis=1) & validw
    vals = jnp.where(rep, S, -jnp.inf)

    cols_l, valid_l = [], []
    cur = vals
    lane = jnp.arange(_WC, dtype=jnp.int32)[None, :]
    for _ in range(_TOPK - 1):
        mx = jnp.max(cur, axis=1)
        idx = jnp.argmax(cur, axis=1)
        col = jnp.take_along_axis(C, idx[:, None], axis=1)[:, 0]
        ok = mx > 0
        cols_l.append(jnp.where(ok, col, 0))
        valid_l.append(ok)
        cur = jnp.where(lane == idx[:, None], -jnp.inf, cur)
    cols7 = jnp.stack(cols_l, 1)
    valid7 = jnp.stack(valid_l, 1)

    d0 = jnp.sum(p * t, axis=1)
    dots = jnp.einsum('nd,nkd->nk', p, t[cols7])
    cnt = jnp.sum(valid7.astype(jnp.float32), axis=0)
    inv = jnp.sum(2.0 - 2.0 * d0) / N
    for i in range(_TOPK - 1):
        inv = inv + jnp.sum(
            jnp.where(valid7[:, i], 2.0 - 2.0 * dots[:, i], 0.0)) / cnt[i]
    inv = inv / _TOPK
    c = (p.T @ t) / N
    cd = jnp.diag(c)
    on = jnp.sum((cd - 1.0) ** 2)
    off = jnp.sum(c ** 2) - jnp.sum(cd ** 2)
    loss = inv + _LAMBD * (on + off)
    return (student, loss)


# speedup vs baseline: 1.0461x; 1.0461x over previous
import functools

import jax
import jax.numpy as jnp
from jax import lax
from jax.experimental import pallas as pl
from jax.experimental.pallas import tpu as pltpu
from jax.experimental.pallas import tpu_sc as plsc

_TOPK = 8
_LAMBD = 0.001
_WC = 256

_N, _E, _D = 10000, 320000, 128
_NC, _NS, _NW = 2, 16, 32
_EPW = _E // _NW
_CCH = 400
_NIT = _EPW // _CCH

_sc_mesh = plsc.VectorSubcoreMesh(core_axis_name="c", subcore_axis_name="s")


@functools.partial(
    pl.kernel,
    out_type=jax.ShapeDtypeStruct((_NC * _N,), jnp.float32),
    scratch_types=[
        pltpu.VMEM((_CCH,), jnp.int32),
        pltpu.VMEM((_CCH,), jnp.float32),
        pltpu.VMEM((624,), jnp.float32),
        pltpu.VMEM_SHARED((_N,), jnp.float32),
    ],
    mesh=_sc_mesh)
def _sc_count(idx_hbm, out_hbm, idxv, onesv, tbuf, accum):
    c = lax.axis_index("c")
    s = lax.axis_index("s")
    wid = s * _NC + c

    def fill(i, _):
        onesv[pl.ds(i * 16, 16)] = jnp.full((16,), 1.0, jnp.float32)
        return 0
    lax.fori_loop(0, _CCH // 16, fill, 0)

    def fillz(i, _):
        tbuf[pl.ds(i * 16, 16)] = jnp.zeros((16,), jnp.float32)
        return 0
    lax.fori_loop(0, 624 // 16, fillz, 0)

    pltpu.sync_copy(tbuf, accum.at[pl.ds(s * 624, 624)])
    @pl.when(s == 0)
    def _():
        pltpu.sync_copy(tbuf.at[pl.ds(0, 16)], accum.at[pl.ds(9984, 16)])
    plsc.subcore_barrier()

    def body(it, _):
        b = wid * _EPW + it * _CCH
        pltpu.sync_copy(idx_hbm.at[pl.ds(b, _CCH)], idxv)
        pltpu.sync_copy(onesv, accum.at[idxv], add=True)
        return 0
    lax.fori_loop(0, _NIT, body, 0)
    plsc.subcore_barrier()

    off = c * _N
    pltpu.sync_copy(accum.at[pl.ds(s * 624, 624)], tbuf)
    pltpu.sync_copy(tbuf, out_hbm.at[pl.ds(off + s * 624, 624)])
    @pl.when(s == 0)
    def _():
        pltpu.sync_copy(accum.at[pl.ds(9984, 16)], tbuf.at[pl.ds(0, 16)])
        pltpu.sync_copy(tbuf.at[pl.ds(0, 16)], out_hbm.at[pl.ds(off + 9984, 16)])


_NH = _N // 2


@functools.partial(
    pl.kernel,
    out_type=jax.ShapeDtypeStruct((2, _NC, _NH, _D), jnp.float32),
    scratch_types=[
        pltpu.VMEM((_CCH,), jnp.int32),
        pltpu.VMEM((_CCH,), jnp.int32),
        pltpu.VMEM((_CCH, _D), jnp.float32),
        pltpu.VMEM_SHARED((_NH + 8, _D), jnp.float32),
        pltpu.SemaphoreType.DMA,
    ],
    mesh=_sc_mesh)
def _sc_msg(g_hbm, src_hbm, dst_hbm, out_hbm,
            srcv, dstv, rowsv, accum, sem):
    c = lax.axis_index("c")
    s = lax.axis_index("s")
    wid = s * _NC + c

    for half in range(2):
        lo = half * _NH
        def fillz(i, _):
            rowsv[i // 8, pl.ds((i % 8) * 16, 16)] = (
                jnp.zeros((16,), jnp.float32))
            return 0
        lax.fori_loop(0, 104 * 8, fillz, 0)

        def zslice(k, _):
            pltpu.sync_copy(rowsv.at[pl.ds(0, 104)],
                            accum.at[pl.ds(s * 312 + k * 104, 104)])
            return 0
        lax.fori_loop(0, 3, zslice, 0)
        @pl.when(s == 0)
        def _():
            pltpu.sync_copy(rowsv.at[pl.ds(0, 16)],
                            accum.at[pl.ds(4992, 16)])
        plsc.subcore_barrier()

        def body(it, _):
            b = wid * _EPW + it * _CCH
            pltpu.sync_copy(src_hbm.at[pl.ds(b, _CCH)], srcv)
            pltpu.sync_copy(dst_hbm.at[pl.ds(b, _CCH)], dstv)

            def remap(i, _):
                d = dstv[pl.ds(i * 16, 16)]
                dstv[pl.ds(i * 16, 16)] = jnp.where(
                    (d >= lo) & (d < lo + _NH), d - lo, _NH)
                return 0
            lax.fori_loop(0, _CCH // 16, remap, 0)
            pltpu.async_copy(g_hbm.at[srcv], rowsv, sem).wait()
            pltpu.sync_copy(rowsv, accum.at[dstv], add=True)
            return 0
        lax.fori_loop(0, _NIT, body, 0)
        plsc.subcore_barrier()

        def oslice(k, _):
            pltpu.sync_copy(accum.at[pl.ds(s * 312 + k * 104, 104)],
                            rowsv.at[pl.ds(0, 104)])
            pltpu.sync_copy(
                rowsv.at[pl.ds(0, 104)],
                out_hbm.at[half].at[c].at[pl.ds(s * 312 + k * 104, 104)])
            return 0
        lax.fori_loop(0, 3, oslice, 0)
        @pl.when(s == 0)
        def _():
            pltpu.sync_copy(accum.at[pl.ds(4992, 8)], rowsv.at[pl.ds(0, 8)])
            pltpu.sync_copy(rowsv.at[pl.ds(0, 8)],
                            out_hbm.at[half].at[c].at[pl.ds(4992, 8)])


def kernel(x, y, edge_index, neighbor_index, W1, b1, bn1_g, bn1_b, a1,
           W2, b2, bn2_g, bn2_b, a2, Wp1, bp1, bnp_g, bnp_b, ap, Wp2, bp2):
    N, D = x.shape
    src, dst = edge_index[0], edge_index[1]
    E = src.shape[0]

    degp = _sc_count(dst)
    deg = degp[:N] + degp[N:] + 1.0
    dinv = jax.lax.rsqrt(deg)

    def layer(xin, W, b, g, beta, a):
        h = xin @ W
        gsc = h * dinv[:, None]
        msgp = _sc_msg(gsc, src, dst)
        msg = jnp.concatenate(
            [msgp[0, 0] + msgp[0, 1], msgp[1, 0] + msgp[1, 1]], axis=0)
        out = dinv[:, None] * msg + (dinv * dinv)[:, None] * h + b
        m = jnp.mean(out, axis=0)
        v = jnp.var(out, axis=0)
        out = (out - m) / jnp.sqrt(v + 1e-5) * g + beta
        return jnp.where(out >= 0, out, a * out)

    x1 = layer(x, W1, b1, bn1_g, bn1_b, a1)
    student = layer(x1, W2, b2, bn2_g, bn2_b, a2)

    h = student @ Wp1 + bp1
    m = jnp.mean(h, axis=0)
    v = jnp.var(h, axis=0)
    h = (h - m) / jnp.sqrt(v + 1e-5) * bnp_g + bnp_b
    h = jnp.where(h >= 0, h, ap * h)
    pred = h @ Wp2 + bp2

    sn = student / jnp.clip(jnp.linalg.norm(student, axis=-1, keepdims=True), 1e-12)
    p = pred / jnp.clip(jnp.linalg.norm(pred, axis=-1, keepdims=True), 1e-12)
    t = sn

    ns, nd = neighbor_index[0], neighbor_index[1]
    key_s = jnp.sort(ns * N + nd)
    ns_s = key_s // N
    nd_s = key_s % N
    sim = jnp.sum(sn[ns_s] * t[nd_s], axis=-1)

    rdegp = _sc_count(ns)
    rdeg = (rdegp[:N] + rdegp[N:]).astype(jnp.int32)
    row_start = jnp.concatenate([jnp.zeros((1,), jnp.int32),
                                 jnp.cumsum(rdeg).astype(jnp.int32)])

    w = jnp.arange(_WC, dtype=jnp.int32)
    flat = row_start[:N, None] + w[None, :]
    validw = w[None, :] < rdeg[:, None]
    flat_c = jnp.minimum(flat, E - 1)
    T = jnp.where(validw, sim[flat_c], 0.0)
    C = jnp.where(validw, nd_s[flat_c], -1)

    S = T
    d_ = 1
    while d_ < _WC:
        C_sh = jnp.concatenate(
            [C[:, d_:], jnp.full((N, d_), -2, C.dtype)], axis=1)
        S_sh = jnp.concatenate(
            [S[:, d_:], jnp.zeros((N, d_), S.dtype)], axis=1)
        S = S + jnp.where(C == C_sh, S_sh, 0.0)
        d_ *= 2

    rep = jnp.concatenate(
        [jnp.ones((N, 1), bool), C[:, 1:] != C[:, :-1]], axis=1) & validw
    vals = jnp.where(rep, S, -jnp.inf)

    cols_l, valid_l = [], []
    cur = vals
    lane = jnp.arange(_WC, dtype=jnp.int32)[None, :]
    for _ in range(_TOPK - 1):
        mx = jnp.max(cur, axis=1)
        idx = jnp.argmax(cur, axis=1)
        col = jnp.take_along_axis(C, idx[:, None], axis=1)[:, 0]
        ok = mx > 0
        cols_l.append(jnp.where(ok, col, 0))
        valid_l.append(ok)
        cur = jnp.where(lane == idx[:, None], -jnp.inf, cur)
    cols7 = jnp.stack(cols_l, 1)
    valid7 = jnp.stack(valid_l, 1)

    d0 = jnp.sum(p * t, axis=1)
    dots = jnp.einsum('nd,nkd->nk', p, t[cols7])
    cnt = jnp.sum(valid7.astype(jnp.float32), axis=0)
    inv = jnp.sum(2.0 - 2.0 * d0) / N
    for i in range(_TOPK - 1):
        inv = inv + jnp.sum(
            jnp.where(valid7[:, i], 2.0 - 2.0 * dots[:, i], 0.0)) / cnt[i]
    inv = inv / _TOPK
    c = (p.T @ t) / N
    cd = jnp.diag(c)
    on = jnp.sum((cd - 1.0) ** 2)
    off = jnp.sum(c ** 2) - jnp.sum(cd ** 2)
    loss = inv + _LAMBD * (on + off)
    return (student, loss)


# speedup vs baseline: 17.9664x; 17.1752x over previous
import functools

import jax
import jax.numpy as jnp
from jax import lax
from jax.experimental import pallas as pl
from jax.experimental.pallas import tpu as pltpu
from jax.experimental.pallas import tpu_sc as plsc

_TOPK = 8
_LAMBD = 0.001
_WC = 256

_N, _E, _D = 10000, 320000, 128
_NC, _NS, _NW = 2, 16, 32
_EPW = _E // _NW
_CCH = 400
_NIT = _EPW // _CCH

_sc_mesh = plsc.VectorSubcoreMesh(core_axis_name="c", subcore_axis_name="s")


@functools.partial(
    pl.kernel,
    out_type=jax.ShapeDtypeStruct((_NC * _N,), jnp.float32),
    scratch_types=[
        pltpu.VMEM((_CCH,), jnp.int32),
        pltpu.VMEM((_CCH,), jnp.float32),
        pltpu.VMEM((624,), jnp.float32),
        pltpu.VMEM_SHARED((_N,), jnp.float32),
    ],
    mesh=_sc_mesh)
def _sc_count(idx_hbm, out_hbm, idxv, onesv, tbuf, accum):
    c = lax.axis_index("c")
    s = lax.axis_index("s")
    wid = s * _NC + c

    def fill(i, _):
        onesv[pl.ds(i * 16, 16)] = jnp.full((16,), 1.0, jnp.float32)
        return 0
    lax.fori_loop(0, _CCH // 16, fill, 0)

    def fillz(i, _):
        tbuf[pl.ds(i * 16, 16)] = jnp.zeros((16,), jnp.float32)
        return 0
    lax.fori_loop(0, 624 // 16, fillz, 0)

    pltpu.sync_copy(tbuf, accum.at[pl.ds(s * 624, 624)])
    @pl.when(s == 0)
    def _():
        pltpu.sync_copy(tbuf.at[pl.ds(0, 16)], accum.at[pl.ds(9984, 16)])
    plsc.subcore_barrier()

    def body(it, _):
        b = wid * _EPW + it * _CCH
        pltpu.sync_copy(idx_hbm.at[pl.ds(b, _CCH)], idxv)
        pltpu.sync_copy(onesv, accum.at[idxv], add=True)
        return 0
    lax.fori_loop(0, _NIT, body, 0)
    plsc.subcore_barrier()

    off = c * _N
    pltpu.sync_copy(accum.at[pl.ds(s * 624, 624)], tbuf)
    pltpu.sync_copy(tbuf, out_hbm.at[pl.ds(off + s * 624, 624)])
    @pl.when(s == 0)
    def _():
        pltpu.sync_copy(accum.at[pl.ds(9984, 16)], tbuf.at[pl.ds(0, 16)])
        pltpu.sync_copy(tbuf.at[pl.ds(0, 16)], out_hbm.at[pl.ds(off + 9984, 16)])


_NPAIR = 81920
_RPW = 313
def _make_pairdot(total, cch):
    epw = total // _NW
    nit = epw // cch

    @functools.partial(
        pl.kernel,
        out_type=jax.ShapeDtypeStruct((total * 16,), jnp.float32),
        scratch_types=[
            pltpu.VMEM((cch,), jnp.int32),
            pltpu.VMEM((cch,), jnp.int32),
            pltpu.VMEM((cch, _D), jnp.float32),
            pltpu.VMEM((cch, _D), jnp.float32),
            pltpu.VMEM((cch * 16,), jnp.float32),
            pltpu.SemaphoreType.DMA,
            pltpu.SemaphoreType.DMA,
        ],
        mesh=_sc_mesh)
    def dotk(ns_hbm, nd_hbm, a_hbm, b_hbm, sim_out,
             nsv, ndv, ra, rb, simv, sema, semb):
        c = lax.axis_index("c")
        s = lax.axis_index("s")
        wid = s * _NC + c

        def body(it, _):
            bofs = wid * epw + it * cch
            pltpu.sync_copy(ns_hbm.at[pl.ds(bofs, cch)], nsv)
            pltpu.sync_copy(nd_hbm.at[pl.ds(bofs, cch)], ndv)
            cpa = pltpu.async_copy(a_hbm.at[nsv], ra, sema)
            cpb = pltpu.async_copy(b_hbm.at[ndv], rb, semb)
            cpa.wait()
            cpb.wait()

            def dote(e, _):
                acc = ra[e, pl.ds(0, 16)] * rb[e, pl.ds(0, 16)]
                for j in range(1, _D // 16):
                    acc = acc + (ra[e, pl.ds(j * 16, 16)]
                                 * rb[e, pl.ds(j * 16, 16)])
                simv[pl.ds(e * 16, 16)] = acc
                return 0
            lax.fori_loop(0, cch, dote, 0)

            pltpu.sync_copy(simv, sim_out.at[pl.ds(bofs * 16, cch * 16)])
            return 0
        lax.fori_loop(0, nit, body, 0)
    return dotk


_sc_edge_sim = _make_pairdot(_E, _CCH)
_sc_pair_dot = _make_pairdot(_NPAIR, 320)


_TPW = _N * _WC // _NW


@functools.partial(
    pl.kernel,
    out_type=[jax.ShapeDtypeStruct((_N * _WC,), jnp.float32),
              jax.ShapeDtypeStruct((_N * _WC,), jnp.int32)],
    scratch_types=[
        pltpu.VMEM((2000,), jnp.int32),
        pltpu.VMEM((2000,), jnp.float32),
        pltpu.VMEM((2000,), jnp.int32),
        pltpu.SemaphoreType.DMA,
        pltpu.SemaphoreType.DMA,
    ],
    mesh=_sc_mesh)
def _sc_table(flat_hbm, sim_hbm, nd_hbm, t_out, c_out,
              idxb, tvb, cvb, sema, semb):
    c = lax.axis_index("c")
    s = lax.axis_index("s")
    wid = s * _NC + c

    def body(it, _):
        b = wid * _TPW + it * 2000
        pltpu.sync_copy(flat_hbm.at[pl.ds(b, 2000)], idxb)
        pltpu.async_copy(sim_hbm.at[idxb], tvb, sema).wait()
        pltpu.async_copy(nd_hbm.at[idxb], cvb, semb).wait()
        pltpu.sync_copy(tvb, t_out.at[pl.ds(b, 2000)])
        pltpu.sync_copy(cvb, c_out.at[pl.ds(b, 2000)])
        return 0
    lax.fori_loop(0, _TPW // 2000, body, 0)


_NH = _N // 2


@functools.partial(
    pl.kernel,
    out_type=jax.ShapeDtypeStruct((2, _NC, _NH, _D), jnp.float32),
    scratch_types=[
        pltpu.VMEM((_CCH,), jnp.int32),
        pltpu.VMEM((_CCH,), jnp.int32),
        pltpu.VMEM((_CCH, _D), jnp.float32),
        pltpu.VMEM_SHARED((_NH + 8, _D), jnp.float32),
        pltpu.SemaphoreType.DMA,
    ],
    mesh=_sc_mesh)
def _sc_msg(g_hbm, src_hbm, dst_hbm, out_hbm,
            srcv, dstv, rowsv, accum, sem):
    c = lax.axis_index("c")
    s = lax.axis_index("s")
    wid = s * _NC + c

    for half in range(2):
        lo = half * _NH
        def fillz(i, _):
            rowsv[i // 8, pl.ds((i % 8) * 16, 16)] = (
                jnp.zeros((16,), jnp.float32))
            return 0
        lax.fori_loop(0, 104 * 8, fillz, 0)

        def zslice(k, _):
            pltpu.sync_copy(rowsv.at[pl.ds(0, 104)],
                            accum.at[pl.ds(s * 312 + k * 104, 104)])
            return 0
        lax.fori_loop(0, 3, zslice, 0)
        @pl.when(s == 0)
        def _():
            pltpu.sync_copy(rowsv.at[pl.ds(0, 16)],
                            accum.at[pl.ds(4992, 16)])
        plsc.subcore_barrier()

        def body(it, _):
            b = wid * _EPW + it * _CCH
            pltpu.sync_copy(src_hbm.at[pl.ds(b, _CCH)], srcv)
            pltpu.sync_copy(dst_hbm.at[pl.ds(b, _CCH)], dstv)

            def remap(i, _):
                d = dstv[pl.ds(i * 16, 16)]
                dstv[pl.ds(i * 16, 16)] = jnp.where(
                    (d >= lo) & (d < lo + _NH), d - lo, _NH)
                return 0
            lax.fori_loop(0, _CCH // 16, remap, 0)
            pltpu.async_copy(g_hbm.at[srcv], rowsv, sem).wait()
            pltpu.sync_copy(rowsv, accum.at[dstv], add=True)
            return 0
        lax.fori_loop(0, _NIT, body, 0)
        plsc.subcore_barrier()

        def oslice(k, _):
            pltpu.sync_copy(accum.at[pl.ds(s * 312 + k * 104, 104)],
                            rowsv.at[pl.ds(0, 104)])
            pltpu.sync_copy(
                rowsv.at[pl.ds(0, 104)],
                out_hbm.at[half].at[c].at[pl.ds(s * 312 + k * 104, 104)])
            return 0
        lax.fori_loop(0, 3, oslice, 0)
        @pl.when(s == 0)
        def _():
            pltpu.sync_copy(accum.at[pl.ds(4992, 8)], rowsv.at[pl.ds(0, 8)])
            pltpu.sync_copy(rowsv.at[pl.ds(0, 8)],
                            out_hbm.at[half].at[c].at[pl.ds(4992, 8)])


def kernel(x, y, edge_index, neighbor_index, W1, b1, bn1_g, bn1_b, a1,
           W2, b2, bn2_g, bn2_b, a2, Wp1, bp1, bnp_g, bnp_b, ap, Wp2, bp2):
    N, D = x.shape
    src, dst = edge_index[0], edge_index[1]
    E = src.shape[0]

    degp = _sc_count(dst)
    deg = degp[:N] + degp[N:] + 1.0
    dinv = jax.lax.rsqrt(deg)

    def layer(xin, W, b, g, beta, a):
        h = xin @ W
        gsc = h * dinv[:, None]
        msgp = _sc_msg(gsc, src, dst)
        msg = jnp.concatenate(
            [msgp[0, 0] + msgp[0, 1], msgp[1, 0] + msgp[1, 1]], axis=0)
        out = dinv[:, None] * msg + (dinv * dinv)[:, None] * h + b
        m = jnp.mean(out, axis=0)
        v = jnp.var(out, axis=0)
        out = (out - m) / jnp.sqrt(v + 1e-5) * g + beta
        return jnp.where(out >= 0, out, a * out)

    x1 = layer(x, W1, b1, bn1_g, bn1_b, a1)
    student = layer(x1, W2, b2, bn2_g, bn2_b, a2)

    h = student @ Wp1 + bp1
    m = jnp.mean(h, axis=0)
    v = jnp.var(h, axis=0)
    h = (h - m) / jnp.sqrt(v + 1e-5) * bnp_g + bnp_b
    h = jnp.where(h >= 0, h, ap * h)
    pred = h @ Wp2 + bp2

    sn = student / jnp.clip(jnp.linalg.norm(student, axis=-1, keepdims=True), 1e-12)
    p = pred / jnp.clip(jnp.linalg.norm(pred, axis=-1, keepdims=True), 1e-12)
    t = sn

    ns, nd = neighbor_index[0], neighbor_index[1]
    key_s = jnp.sort(ns * N + nd)
    ns_s = key_s // N
    nd_s = key_s - ns_s * N
    sim_s = _sc_edge_sim(ns_s, nd_s, sn, t).reshape(E, 16).sum(axis=1)

    rdegp = _sc_count(ns)
    rdeg = (rdegp[:N] + rdegp[N:]).astype(jnp.int32)
    row_start = jnp.concatenate([jnp.zeros((1,), jnp.int32),
                                 jnp.cumsum(rdeg).astype(jnp.int32)])
    w = jnp.arange(_WC, dtype=jnp.int32)
    flatidx = jnp.minimum(
        row_start[:N, None] + w[None, :], E - 1).reshape(-1)
    t_flat, c_flat = _sc_table(flatidx, sim_s, nd_s)
    validw = w[None, :] < rdeg[:, None]
    T = jnp.where(validw, t_flat.reshape(N, _WC), 0.0)
    C = jnp.where(validw, c_flat.reshape(N, _WC), -1)

    S = T
    d_ = 1
    while d_ < _WC:
        C_sh = jnp.concatenate(
            [C[:, d_:], jnp.full((N, d_), -2, C.dtype)], axis=1)
        S_sh = jnp.concatenate(
            [S[:, d_:], jnp.zeros((N, d_), S.dtype)], axis=1)
        S = S + jnp.where(C == C_sh, S_sh, 0.0)
        d_ *= 2

    rep = jnp.concatenate(
        [jnp.ones((N, 1), bool), C[:, 1:] != C[:, :-1]], axis=1) & validw
    vals = jnp.where(rep, S, -jnp.inf)

    cols_l, valid_l = [], []
    cur = vals
    lane = jnp.arange(_WC, dtype=jnp.int32)[None, :]
    for _ in range(_TOPK - 1):
        mx = jnp.max(cur, axis=1)
        idx = jnp.argmax(cur, axis=1)
        col = jnp.take_along_axis(C, idx[:, None], axis=1)[:, 0]
        ok = mx > 0
        cols_l.append(jnp.where(ok, col, 0))
        valid_l.append(ok)
        cur = jnp.where(lane == idx[:, None], -jnp.inf, cur)
    cols7 = jnp.stack(cols_l, 1)
    valid7 = jnp.stack(valid_l, 1)

    d0 = jnp.sum(p * t, axis=1)
    cols8 = jnp.concatenate(
        [cols7, jnp.zeros((N, 1), jnp.int32)], axis=1)
    cols_pad = jnp.concatenate(
        [cols8, jnp.zeros((_NPAIR // 8 - N, 8), jnp.int32)], axis=0)
    rowidx = jnp.minimum(jnp.arange(_NPAIR, dtype=jnp.int32) // 8, N - 1)
    dots_flat = _sc_pair_dot(
        rowidx, cols_pad.reshape(-1), p, t).reshape(_NPAIR, 16).sum(axis=1)
    dots = dots_flat.reshape(_NPAIR // 8, 8)[:N, :_TOPK - 1]
    cnt = jnp.sum(valid7.astype(jnp.float32), axis=0)
    inv = jnp.sum(2.0 - 2.0 * d0) / N
    for i in range(_TOPK - 1):
        inv = inv + jnp.sum(
            jnp.where(valid7[:, i], 2.0 - 2.0 * dots[:, i], 0.0)) / cnt[i]
    inv = inv / _TOPK
    c = (p.T @ t) / N
    cd = jnp.diag(c)
    on = jnp.sum((cd - 1.0) ** 2)
    off = jnp.sum(c ** 2) - jnp.sum(cd ** 2)
    loss = inv + _LAMBD * (on + off)
    return (student, loss)


# speedup vs baseline: 19.9173x; 1.1086x over previous
import functools

import jax
import jax.numpy as jnp
from jax import lax
from jax.experimental import pallas as pl
from jax.experimental.pallas import tpu as pltpu
from jax.experimental.pallas import tpu_sc as plsc

_TOPK = 8
_LAMBD = 0.001
_WC = 256

_N, _E, _D = 10000, 320000, 128
_NC, _NS, _NW = 2, 16, 32
_EPW = _E // _NW
_CCH = 400
_NIT = _EPW // _CCH

_sc_mesh = plsc.VectorSubcoreMesh(core_axis_name="c", subcore_axis_name="s")


@functools.partial(
    pl.kernel,
    out_type=jax.ShapeDtypeStruct((_NC * _N,), jnp.float32),
    scratch_types=[
        pltpu.VMEM((_CCH,), jnp.int32),
        pltpu.VMEM((_CCH,), jnp.float32),
        pltpu.VMEM((624,), jnp.float32),
        pltpu.VMEM_SHARED((_N,), jnp.float32),
    ],
    mesh=_sc_mesh)
def _sc_count(idx_hbm, out_hbm, idxv, onesv, tbuf, accum):
    c = lax.axis_index("c")
    s = lax.axis_index("s")
    wid = s * _NC + c

    def fill(i, _):
        onesv[pl.ds(i * 16, 16)] = jnp.full((16,), 1.0, jnp.float32)
        return 0
    lax.fori_loop(0, _CCH // 16, fill, 0)

    def fillz(i, _):
        tbuf[pl.ds(i * 16, 16)] = jnp.zeros((16,), jnp.float32)
        return 0
    lax.fori_loop(0, 624 // 16, fillz, 0)

    pltpu.sync_copy(tbuf, accum.at[pl.ds(s * 624, 624)])
    @pl.when(s == 0)
    def _():
        pltpu.sync_copy(tbuf.at[pl.ds(0, 16)], accum.at[pl.ds(9984, 16)])
    plsc.subcore_barrier()

    def body(it, _):
        b = wid * _EPW + it * _CCH
        pltpu.sync_copy(idx_hbm.at[pl.ds(b, _CCH)], idxv)
        pltpu.sync_copy(onesv, accum.at[idxv], add=True)
        return 0
    lax.fori_loop(0, _NIT, body, 0)
    plsc.subcore_barrier()

    off = c * _N
    pltpu.sync_copy(accum.at[pl.ds(s * 624, 624)], tbuf)
    pltpu.sync_copy(tbuf, out_hbm.at[pl.ds(off + s * 624, 624)])
    @pl.when(s == 0)
    def _():
        pltpu.sync_copy(accum.at[pl.ds(9984, 16)], tbuf.at[pl.ds(0, 16)])
        pltpu.sync_copy(tbuf.at[pl.ds(0, 16)], out_hbm.at[pl.ds(off + 9984, 16)])


_NPAIR = 81920
_RPW = 313
def _make_pairdot(total, cch):
    epw = total // _NW
    nit = epw // cch

    @functools.partial(
        pl.kernel,
        out_type=jax.ShapeDtypeStruct((total * 16,), jnp.float32),
        scratch_types=[
            pltpu.VMEM((cch,), jnp.int32),
            pltpu.VMEM((cch,), jnp.int32),
            pltpu.VMEM((cch, _D), jnp.float32),
            pltpu.VMEM((cch, _D), jnp.float32),
            pltpu.VMEM((cch * 16,), jnp.float32),
            pltpu.SemaphoreType.DMA,
            pltpu.SemaphoreType.DMA,
        ],
        mesh=_sc_mesh)
    def dotk(ns_hbm, nd_hbm, a_hbm, b_hbm, sim_out,
             nsv, ndv, ra, rb, simv, sema, semb):
        c = lax.axis_index("c")
        s = lax.axis_index("s")
        wid = s * _NC + c

        def body(it, _):
            bofs = wid * epw + it * cch
            pltpu.sync_copy(ns_hbm.at[pl.ds(bofs, cch)], nsv)
            pltpu.sync_copy(nd_hbm.at[pl.ds(bofs, cch)], ndv)
            cpa = pltpu.async_copy(a_hbm.at[nsv], ra, sema)
            cpb = pltpu.async_copy(b_hbm.at[ndv], rb, semb)
            cpa.wait()
            cpb.wait()

            def dote(e, _):
                acc = ra[e, pl.ds(0, 16)] * rb[e, pl.ds(0, 16)]
                for j in range(1, _D // 16):
                    acc = acc + (ra[e, pl.ds(j * 16, 16)]
                                 * rb[e, pl.ds(j * 16, 16)])
                simv[pl.ds(e * 16, 16)] = acc
                return 0
            lax.fori_loop(0, cch, dote, 0)

            pltpu.sync_copy(simv, sim_out.at[pl.ds(bofs * 16, cch * 16)])
            return 0
        lax.fori_loop(0, nit, body, 0)
    return dotk


_sc_edge_sim = _make_pairdot(_E, _CCH)
_sc_pair_dot = _make_pairdot(_NPAIR, 320)


_TPW = _N * _WC // _NW


@functools.partial(
    pl.kernel,
    out_type=[jax.ShapeDtypeStruct((_N * _WC,), jnp.float32),
              jax.ShapeDtypeStruct((_N * _WC,), jnp.int32)],
    scratch_types=[
        pltpu.VMEM((2000,), jnp.int32),
        pltpu.VMEM((2000,), jnp.float32),
        pltpu.VMEM((2000,), jnp.int32),
        pltpu.SemaphoreType.DMA,
        pltpu.SemaphoreType.DMA,
    ],
    mesh=_sc_mesh)
def _sc_table(flat_hbm, sim_hbm, nd_hbm, t_out, c_out,
              idxb, tvb, cvb, sema, semb):
    c = lax.axis_index("c")
    s = lax.axis_index("s")
    wid = s * _NC + c

    def body(it, _):
        b = wid * _TPW + it * 2000
        pltpu.sync_copy(flat_hbm.at[pl.ds(b, 2000)], idxb)
        pltpu.async_copy(sim_hbm.at[idxb], tvb, sema).wait()
        pltpu.async_copy(nd_hbm.at[idxb], cvb, semb).wait()
        pltpu.sync_copy(tvb, t_out.at[pl.ds(b, 2000)])
        pltpu.sync_copy(cvb, c_out.at[pl.ds(b, 2000)])
        return 0
    lax.fori_loop(0, _TPW // 2000, body, 0)


_NEG = -1e30


def _bn_prelu(out, g, beta, a):
    m = jnp.mean(out, axis=0)
    d = out - m
    v = jnp.mean(d * d, axis=0)
    y = d / jnp.sqrt(v + 1e-5) * g + beta
    return jnp.where(y >= 0, y, a * y)


def _tc_pre_body(x_ref, w_ref, dv_ref, g_ref, h_ref):
    h = jnp.dot(x_ref[...], w_ref[...], preferred_element_type=jnp.float32)
    h_ref[...] = h
    g_ref[...] = h * dv_ref[...]


def _tc_pre(x, W1, dinv_bc):
    return pl.pallas_call(
        _tc_pre_body,
        out_shape=[jax.ShapeDtypeStruct((_N, _D), jnp.float32),
                   jax.ShapeDtypeStruct((_N, _D), jnp.float32)])(
        x, W1, dinv_bc)


def _tc_mid_body(msg_ref, h_ref, dv_ref, b_ref, g_ref, be_ref, a_ref, w_ref,
                 g2_ref, h2_ref):
    dv = dv_ref[...]
    out = dv * msg_ref[...] + dv * dv * h_ref[...] + b_ref[...]
    x1 = _bn_prelu(out, g_ref[...], be_ref[...], a_ref[0, 0])
    h2 = jnp.dot(x1, w_ref[...], preferred_element_type=jnp.float32)
    h2_ref[...] = h2
    g2_ref[...] = h2 * dv


def _tc_mid(msgsum, h1, dinv_bc, b1, bn_g, bn_b, a1, W2):
    return pl.pallas_call(
        _tc_mid_body,
        out_shape=[jax.ShapeDtypeStruct((_N, _D), jnp.float32),
                   jax.ShapeDtypeStruct((_N, _D), jnp.float32)])(
        msgsum, h1, dinv_bc, b1, bn_g, bn_b, a1, W2)


def _l2n(v):
    n = jnp.sqrt(jnp.sum(v * v, axis=1, keepdims=True))
    return v / jnp.maximum(n, 1e-12)


def _tc_post_body(msg_ref, h_ref, dv_ref, b_ref, g_ref, be_ref, a_ref,
                  wp1_ref, bp1_ref, gp_ref, bep_ref, ap_ref, wp2_ref,
                  bp2_ref, stu_ref, sn_ref, p_ref):
    dv = dv_ref[...]
    out = dv * msg_ref[...] + dv * dv * h_ref[...] + b_ref[...]
    x2 = _bn_prelu(out, g_ref[...], be_ref[...], a_ref[0, 0])
    stu_ref[...] = x2
    hp = jnp.dot(x2, wp1_ref[...],
                 preferred_element_type=jnp.float32) + bp1_ref[...]
    hp = _bn_prelu(hp, gp_ref[...], bep_ref[...], ap_ref[0, 0])
    pred = jnp.dot(hp, wp2_ref[...],
                   preferred_element_type=jnp.float32) + bp2_ref[...]
    sn_ref[...] = _l2n(x2)
    p_ref[...] = _l2n(pred)


def _tc_post(msgsum, h2, dinv_bc, b2, bn_g, bn_b, a2,
             Wp1, bp1, bnp_g, bnp_b, ap, Wp2, bp2):
    return pl.pallas_call(
        _tc_post_body,
        out_shape=[jax.ShapeDtypeStruct((_N, _D), jnp.float32),
                   jax.ShapeDtypeStruct((_N, _D), jnp.float32),
                   jax.ShapeDtypeStruct((_N, _D), jnp.float32)])(
        msgsum, h2, dinv_bc, b2, bn_g, bn_b, a2,
        Wp1, bp1, bnp_g, bnp_b, ap, Wp2, bp2)


_RMINE = 2000


def _tc_mine_body(t_ref, c_ref, o_ref):
    T = t_ref[...]
    C = c_ref[...]
    R = T.shape[0]
    S = T
    d_ = 1
    while d_ < _WC:
        csh = jnp.concatenate(
            [C[:, d_:], jnp.full((R, d_), -2, C.dtype)], axis=1)
        ssh = jnp.concatenate(
            [S[:, d_:], jnp.zeros((R, d_), S.dtype)], axis=1)
        S = S + jnp.where(C == csh, ssh, 0.0)
        d_ *= 2
    cprev = jnp.concatenate(
        [jnp.full((R, 1), -3, C.dtype), C[:, :-1]], axis=1)
    rep = (C != cprev) & (C >= 0)
    cur = jnp.where(rep, S, _NEG)
    iota2 = lax.broadcasted_iota(jnp.int32, (R, _WC), 1)
    li = lax.broadcasted_iota(jnp.int32, (R, _D), 1)
    Cf = C.astype(jnp.float32)
    out = jnp.zeros((R, _D), jnp.float32)
    for i in range(_TOPK - 1):
        mx = jnp.max(cur, axis=1, keepdims=True)
        sel = cur == mx
        firstw = jnp.min(jnp.where(sel, iota2, _WC), axis=1, keepdims=True)
        onehot = iota2 == firstw
        colf = jnp.sum(jnp.where(onehot, Cf, 0.0), axis=1)
        ok = (mx[:, 0] > 0).astype(jnp.float32)
        colf = colf * ok
        out = jnp.where(li == i, colf[:, None], out)
        out = jnp.where(li == 8 + i, ok[:, None], out)
        cur = jnp.where(onehot, _NEG, cur)
    o_ref[...] = out


def _tc_mine(T, C):
    return pl.pallas_call(
        _tc_mine_body,
        grid=(_N // _RMINE,),
        in_specs=[pl.BlockSpec((_RMINE, _WC), lambda i: (i, 0)),
                  pl.BlockSpec((_RMINE, _WC), lambda i: (i, 0))],
        out_specs=pl.BlockSpec((_RMINE, _D), lambda i: (i, 0)),
        out_shape=jax.ShapeDtypeStruct((_N, _D), jnp.float32))(T, C)


def _tc_loss_body(p_ref, t_ref, pd_ref, mine_ref, o_ref):
    p_ = p_ref[...]
    t_ = t_ref[...]
    pd = pd_ref[...]
    mn = mine_ref[...]
    li = lax.broadcasted_iota(jnp.int32, (_N, _D), 1)
    d0 = jnp.sum(p_ * t_, axis=1)
    inv = jnp.sum(2.0 - 2.0 * d0) / _N
    for i in range(_TOPK - 1):
        dots_i = jnp.sum(pd[:_N, i * 16:(i + 1) * 16], axis=1)
        ok_i = jnp.sum(jnp.where(li == 8 + i, mn, 0.0), axis=1)
        cnt = jnp.sum(ok_i)
        inv = inv + jnp.sum(ok_i * (2.0 - 2.0 * dots_i)) / cnt
    inv = inv / _TOPK
    c = lax.dot_general(p_, t_, (((0,), (0,)), ((), ())),
                        preferred_element_type=jnp.float32) / _N
    eye = (lax.broadcasted_iota(jnp.int32, (_D, _D), 0)
           == lax.broadcasted_iota(jnp.int32, (_D, _D), 1))
    on = jnp.sum(jnp.where(eye, (c - 1.0) ** 2, 0.0))
    off = jnp.sum(jnp.where(eye, 0.0, c * c))
    o_ref[...] = jnp.reshape(inv + _LAMBD * (on + off), (1, 1))


def _tc_loss(p, t, pd2d, mined):
    return pl.pallas_call(
        _tc_loss_body,
        out_shape=jax.ShapeDtypeStruct((1, 1), jnp.float32))(
        p, t, pd2d, mined)


_NH = _N // 2


@functools.partial(
    pl.kernel,
    out_type=jax.ShapeDtypeStruct((2, _NC, _NH, _D), jnp.float32),
    scratch_types=[
        pltpu.VMEM((_CCH,), jnp.int32),
        pltpu.VMEM((_CCH,), jnp.int32),
        pltpu.VMEM((_CCH, _D), jnp.float32),
        pltpu.VMEM_SHARED((_NH + 8, _D), jnp.float32),
        pltpu.SemaphoreType.DMA,
    ],
    mesh=_sc_mesh)
def _sc_msg(g_hbm, src_hbm, dst_hbm, out_hbm,
            srcv, dstv, rowsv, accum, sem):
    c = lax.axis_index("c")
    s = lax.axis_index("s")
    wid = s * _NC + c

    for half in range(2):
        lo = half * _NH
        def fillz(i, _):
            rowsv[i // 8, pl.ds((i % 8) * 16, 16)] = (
                jnp.zeros((16,), jnp.float32))
            return 0
        lax.fori_loop(0, 104 * 8, fillz, 0)

        def zslice(k, _):
            pltpu.sync_copy(rowsv.at[pl.ds(0, 104)],
                            accum.at[pl.ds(s * 312 + k * 104, 104)])
            return 0
        lax.fori_loop(0, 3, zslice, 0)
        @pl.when(s == 0)
        def _():
            pltpu.sync_copy(rowsv.at[pl.ds(0, 16)],
                            accum.at[pl.ds(4992, 16)])
        plsc.subcore_barrier()

        def body(it, _):
            b = wid * _EPW + it * _CCH
            pltpu.sync_copy(src_hbm.at[pl.ds(b, _CCH)], srcv)
            pltpu.sync_copy(dst_hbm.at[pl.ds(b, _CCH)], dstv)

            def remap(i, _):
                d = dstv[pl.ds(i * 16, 16)]
                dstv[pl.ds(i * 16, 16)] = jnp.where(
                    (d >= lo) & (d < lo + _NH), d - lo, _NH)
                return 0
            lax.fori_loop(0, _CCH // 16, remap, 0)
            pltpu.async_copy(g_hbm.at[srcv], rowsv, sem).wait()
            pltpu.sync_copy(rowsv, accum.at[dstv], add=True)
            return 0
        lax.fori_loop(0, _NIT, body, 0)
        plsc.subcore_barrier()

        def oslice(k, _):
            pltpu.sync_copy(accum.at[pl.ds(s * 312 + k * 104, 104)],
                            rowsv.at[pl.ds(0, 104)])
            pltpu.sync_copy(
                rowsv.at[pl.ds(0, 104)],
                out_hbm.at[half].at[c].at[pl.ds(s * 312 + k * 104, 104)])
            return 0
        lax.fori_loop(0, 3, oslice, 0)
        @pl.when(s == 0)
        def _():
            pltpu.sync_copy(accum.at[pl.ds(4992, 8)], rowsv.at[pl.ds(0, 8)])
            pltpu.sync_copy(rowsv.at[pl.ds(0, 8)],
                            out_hbm.at[half].at[c].at[pl.ds(4992, 8)])


def kernel(x, y, edge_index, neighbor_index, W1, b1, bn1_g, bn1_b, a1,
           W2, b2, bn2_g, bn2_b, a2, Wp1, bp1, bnp_g, bnp_b, ap, Wp2, bp2):
    N, D = x.shape
    src, dst = edge_index[0], edge_index[1]
    E = src.shape[0]

    degp = _sc_count(dst)
    dinv = jax.lax.rsqrt(degp[:N] + degp[N:] + 1.0)
    dinv_bc = jnp.broadcast_to(dinv[:, None], (N, D))

    def _merge(msgp):
        return jnp.concatenate(
            [msgp[0, 0] + msgp[0, 1], msgp[1, 0] + msgp[1, 1]], axis=0)

    r1 = b1.reshape(1, D)
    g1, h1 = _tc_pre(x, W1, dinv_bc)
    msgsum1 = _merge(_sc_msg(g1, src, dst))
    g2, h2 = _tc_mid(msgsum1, h1, dinv_bc, r1, bn1_g.reshape(1, D),
                     bn1_b.reshape(1, D), a1.reshape(1, 1), W2)
    msgsum2 = _merge(_sc_msg(g2, src, dst))
    student, sn, p = _tc_post(
        msgsum2, h2, dinv_bc, b2.reshape(1, D), bn2_g.reshape(1, D),
        bn2_b.reshape(1, D), a2.reshape(1, 1), Wp1, bp1.reshape(1, -1),
        bnp_g.reshape(1, -1), bnp_b.reshape(1, -1), ap.reshape(1, 1),
        Wp2, bp2.reshape(1, D))
    t = sn

    ns, nd = neighbor_index[0], neighbor_index[1]
    key_s = jnp.sort(ns * N + nd)
    ns_s = key_s // N
    nd_s = key_s - ns_s * N
    sim_s = _sc_edge_sim(ns_s, nd_s, sn, t).reshape(E, 16).sum(axis=1)

    rdegp = _sc_count(ns)
    rdeg = (rdegp[:N] + rdegp[N:]).astype(jnp.int32)
    row_start = jnp.concatenate([jnp.zeros((1,), jnp.int32),
                                 jnp.cumsum(rdeg).astype(jnp.int32)])
    w = jnp.arange(_WC, dtype=jnp.int32)
    flatidx = jnp.minimum(
        row_start[:N, None] + w[None, :], E - 1).reshape(-1)
    t_flat, c_flat = _sc_table(flatidx, sim_s, nd_s)
    validw = w[None, :] < rdeg[:, None]
    T = jnp.where(validw, t_flat.reshape(N, _WC), 0.0)
    C = jnp.where(validw, c_flat.reshape(N, _WC), -1)

    mined = _tc_mine(T, C)
    cols7 = mined[:, :_TOPK - 1].astype(jnp.int32)

    cols8 = jnp.concatenate(
        [cols7, jnp.zeros((N, 1), jnp.int32)], axis=1)
    cols_pad = jnp.concatenate(
        [cols8, jnp.zeros((_NPAIR // 8 - N, 8), jnp.int32)], axis=0)
    rowidx = jnp.minimum(jnp.arange(_NPAIR, dtype=jnp.int32) // 8, N - 1)
    pd2d = _sc_pair_dot(
        rowidx, cols_pad.reshape(-1), p, t).reshape(_NPAIR // 8, 8 * 16)
    loss = _tc_loss(p, t, pd2d, mined)[0, 0]
    return (student, loss)


# speedup vs baseline: 20.5518x; 1.0319x over previous
import functools

import jax
import jax.numpy as jnp
from jax import lax
from jax.experimental import pallas as pl
from jax.experimental.pallas import tpu as pltpu
from jax.experimental.pallas import tpu_sc as plsc

_TOPK = 8
_LAMBD = 0.001
_WC = 256

_N, _E, _D = 10000, 320000, 128
_NC, _NS, _NW = 2, 16, 32
_EPW = _E // _NW
_CCH = 400
_NIT = _EPW // _CCH

_sc_mesh = plsc.VectorSubcoreMesh(core_axis_name="c", subcore_axis_name="s")


@functools.partial(
    pl.kernel,
    out_type=jax.ShapeDtypeStruct((_NC * _N,), jnp.float32),
    scratch_types=[
        pltpu.VMEM((_CCH,), jnp.int32),
        pltpu.VMEM((_CCH,), jnp.float32),
        pltpu.VMEM((624,), jnp.float32),
        pltpu.VMEM_SHARED((_N,), jnp.float32),
    ],
    mesh=_sc_mesh)
def _sc_count(idx_hbm, out_hbm, idxv, onesv, tbuf, accum):
    c = lax.axis_index("c")
    s = lax.axis_index("s")
    wid = s * _NC + c

    def fill(i, _):
        onesv[pl.ds(i * 16, 16)] = jnp.full((16,), 1.0, jnp.float32)
        return 0
    lax.fori_loop(0, _CCH // 16, fill, 0)

    def fillz(i, _):
        tbuf[pl.ds(i * 16, 16)] = jnp.zeros((16,), jnp.float32)
        return 0
    lax.fori_loop(0, 624 // 16, fillz, 0)

    pltpu.sync_copy(tbuf, accum.at[pl.ds(s * 624, 624)])
    @pl.when(s == 0)
    def _():
        pltpu.sync_copy(tbuf.at[pl.ds(0, 16)], accum.at[pl.ds(9984, 16)])
    plsc.subcore_barrier()

    def body(it, _):
        b = wid * _EPW + it * _CCH
        pltpu.sync_copy(idx_hbm.at[pl.ds(b, _CCH)], idxv)
        pltpu.sync_copy(onesv, accum.at[idxv], add=True)
        return 0
    lax.fori_loop(0, _NIT, body, 0)
    plsc.subcore_barrier()

    off = c * _N
    pltpu.sync_copy(accum.at[pl.ds(s * 624, 624)], tbuf)
    pltpu.sync_copy(tbuf, out_hbm.at[pl.ds(off + s * 624, 624)])
    @pl.when(s == 0)
    def _():
        pltpu.sync_copy(accum.at[pl.ds(9984, 16)], tbuf.at[pl.ds(0, 16)])
        pltpu.sync_copy(tbuf.at[pl.ds(0, 16)], out_hbm.at[pl.ds(off + 9984, 16)])


_NPAIR = 81920
_RPW = 313
def _make_pairdot(total, cch):
    epw = total // _NW
    nit = epw // cch

    @functools.partial(
        pl.kernel,
        out_type=jax.ShapeDtypeStruct((total * 16,), jnp.float32),
        scratch_types=[
            pltpu.VMEM((cch,), jnp.int32),
            pltpu.VMEM((cch,), jnp.int32),
            pltpu.VMEM((cch, _D), jnp.float32),
            pltpu.VMEM((cch, _D), jnp.float32),
            pltpu.VMEM((cch * 16,), jnp.float32),
            pltpu.SemaphoreType.DMA,
            pltpu.SemaphoreType.DMA,
        ],
        mesh=_sc_mesh)
    def dotk(ns_hbm, nd_hbm, a_hbm, b_hbm, sim_out,
             nsv, ndv, ra, rb, simv, sema, semb):
        c = lax.axis_index("c")
        s = lax.axis_index("s")
        wid = s * _NC + c

        def body(it, _):
            bofs = wid * epw + it * cch
            pltpu.sync_copy(ns_hbm.at[pl.ds(bofs, cch)], nsv)
            pltpu.sync_copy(nd_hbm.at[pl.ds(bofs, cch)], ndv)
            cpa = pltpu.async_copy(a_hbm.at[nsv], ra, sema)
            cpb = pltpu.async_copy(b_hbm.at[ndv], rb, semb)
            cpa.wait()
            cpb.wait()

            def dote(e, _):
                acc = ra[e, pl.ds(0, 16)] * rb[e, pl.ds(0, 16)]
                for j in range(1, _D // 16):
                    acc = acc + (ra[e, pl.ds(j * 16, 16)]
                                 * rb[e, pl.ds(j * 16, 16)])
                simv[pl.ds(e * 16, 16)] = acc
                return 0
            lax.fori_loop(0, cch, dote, 0)

            pltpu.sync_copy(simv, sim_out.at[pl.ds(bofs * 16, cch * 16)])
            return 0
        lax.fori_loop(0, nit, body, 0)
    return dotk


def _make_pairdot_db(total, cch):
    epw = total // _NW
    nit = epw // cch

    @functools.partial(
        pl.kernel,
        out_type=jax.ShapeDtypeStruct((total * 16,), jnp.float32),
        scratch_types=[
            pltpu.VMEM((cch,), jnp.int32),
            pltpu.VMEM((cch,), jnp.int32),
            pltpu.VMEM((cch,), jnp.int32),
            pltpu.VMEM((cch,), jnp.int32),
            pltpu.VMEM((cch, _D), jnp.float32),
            pltpu.VMEM((cch, _D), jnp.float32),
            pltpu.VMEM((cch, _D), jnp.float32),
            pltpu.VMEM((cch, _D), jnp.float32),
            pltpu.VMEM((cch * 16,), jnp.float32),
            pltpu.SemaphoreType.DMA,
            pltpu.SemaphoreType.DMA,
            pltpu.SemaphoreType.DMA,
            pltpu.SemaphoreType.DMA,
        ],
        mesh=_sc_mesh)
    def dotk(ns_hbm, nd_hbm, a_hbm, b_hbm, sim_out,
             nsv0, ndv0, nsv1, ndv1, ra0, rb0, ra1, rb1, simv,
             sa0, sb0, sa1, sb1):
        c = lax.axis_index("c")
        s = lax.axis_index("s")
        wid = s * _NC + c
        bufs = ((nsv0, ndv0, ra0, rb0, sa0, sb0),
                (nsv1, ndv1, ra1, rb1, sa1, sb1))

        def launch(i, bi):
            nsv, ndv, ra, rb, sa, sb = bufs[bi]
            b = wid * epw + i * cch
            pltpu.sync_copy(ns_hbm.at[pl.ds(b, cch)], nsv)
            pltpu.sync_copy(nd_hbm.at[pl.ds(b, cch)], ndv)
            pltpu.async_copy(a_hbm.at[nsv], ra, sa)
            pltpu.async_copy(b_hbm.at[ndv], rb, sb)

        def finish(i, bi):
            nsv, ndv, ra, rb, sa, sb = bufs[bi]
            pltpu.make_async_copy(a_hbm.at[nsv], ra, sa).wait()
            pltpu.make_async_copy(b_hbm.at[ndv], rb, sb).wait()

            def dote(e, _):
                acc = ra[e, pl.ds(0, 16)] * rb[e, pl.ds(0, 16)]
                for j in range(1, _D // 16):
                    acc = acc + (ra[e, pl.ds(j * 16, 16)]
                                 * rb[e, pl.ds(j * 16, 16)])
                simv[pl.ds(e * 16, 16)] = acc
                return 0
            lax.fori_loop(0, cch, dote, 0)
            b = wid * epw + i * cch
            pltpu.sync_copy(simv, sim_out.at[pl.ds(b * 16, cch * 16)])

        launch(0, 0)

        def body(k, _):
            i0 = k * 2
            launch(i0 + 1, 1)
            finish(i0, 0)

            @pl.when(i0 + 2 < nit)
            def _():
                launch(i0 + 2, 0)
            finish(i0 + 1, 1)
            return 0
        lax.fori_loop(0, nit // 2, body, 0)
    return dotk


_sc_edge_sim = _make_pairdot_db(_E, 200)
_sc_pair_dot = _make_pairdot(_NPAIR, 320)


_TPW = _N * _WC // _NW


@functools.partial(
    pl.kernel,
    out_type=[jax.ShapeDtypeStruct((_N * _WC,), jnp.float32),
              jax.ShapeDtypeStruct((_N * _WC,), jnp.int32)],
    scratch_types=[
        pltpu.VMEM((2000,), jnp.int32),
        pltpu.VMEM((2000,), jnp.float32),
        pltpu.VMEM((2000,), jnp.int32),
        pltpu.SemaphoreType.DMA,
        pltpu.SemaphoreType.DMA,
    ],
    mesh=_sc_mesh)
def _sc_table(flat_hbm, sim_hbm, nd_hbm, t_out, c_out,
              idxb, tvb, cvb, sema, semb):
    c = lax.axis_index("c")
    s = lax.axis_index("s")
    wid = s * _NC + c

    def body(it, _):
        b = wid * _TPW + it * 2000
        pltpu.sync_copy(flat_hbm.at[pl.ds(b, 2000)], idxb)
        pltpu.async_copy(sim_hbm.at[idxb], tvb, sema).wait()
        pltpu.async_copy(nd_hbm.at[idxb], cvb, semb).wait()
        pltpu.sync_copy(tvb, t_out.at[pl.ds(b, 2000)])
        pltpu.sync_copy(cvb, c_out.at[pl.ds(b, 2000)])
        return 0
    lax.fori_loop(0, _TPW // 2000, body, 0)


_NEG = -1e30


def _bn_prelu(out, g, beta, a):
    m = jnp.mean(out, axis=0)
    d = out - m
    v = jnp.mean(d * d, axis=0)
    y = d / jnp.sqrt(v + 1e-5) * g + beta
    return jnp.where(y >= 0, y, a * y)


def _tc_pre_body(x_ref, w_ref, dv_ref, g_ref, h_ref):
    h = jnp.dot(x_ref[...], w_ref[...], preferred_element_type=jnp.float32)
    h_ref[...] = h
    g_ref[...] = h * dv_ref[...]


def _tc_pre(x, W1, dinv_bc):
    return pl.pallas_call(
        _tc_pre_body,
        out_shape=[jax.ShapeDtypeStruct((_N, _D), jnp.float32),
                   jax.ShapeDtypeStruct((_N, _D), jnp.float32)])(
        x, W1, dinv_bc)


def _tc_mid_body(msg_ref, h_ref, dv_ref, b_ref, g_ref, be_ref, a_ref, w_ref,
                 g2_ref, h2_ref):
    dv = dv_ref[...]
    out = dv * msg_ref[...] + dv * dv * h_ref[...] + b_ref[...]
    x1 = _bn_prelu(out, g_ref[...], be_ref[...], a_ref[0, 0])
    h2 = jnp.dot(x1, w_ref[...], preferred_element_type=jnp.float32)
    h2_ref[...] = h2
    g2_ref[...] = h2 * dv


def _tc_mid(msgsum, h1, dinv_bc, b1, bn_g, bn_b, a1, W2):
    return pl.pallas_call(
        _tc_mid_body,
        out_shape=[jax.ShapeDtypeStruct((_N, _D), jnp.float32),
                   jax.ShapeDtypeStruct((_N, _D), jnp.float32)])(
        msgsum, h1, dinv_bc, b1, bn_g, bn_b, a1, W2)


def _l2n(v):
    n = jnp.sqrt(jnp.sum(v * v, axis=1, keepdims=True))
    return v / jnp.maximum(n, 1e-12)


def _tc_post_body(msg_ref, h_ref, dv_ref, b_ref, g_ref, be_ref, a_ref,
                  wp1_ref, bp1_ref, gp_ref, bep_ref, ap_ref, wp2_ref,
                  bp2_ref, stu_ref, sn_ref, p_ref):
    dv = dv_ref[...]
    out = dv * msg_ref[...] + dv * dv * h_ref[...] + b_ref[...]
    x2 = _bn_prelu(out, g_ref[...], be_ref[...], a_ref[0, 0])
    stu_ref[...] = x2
    hp = jnp.dot(x2, wp1_ref[...],
                 preferred_element_type=jnp.float32) + bp1_ref[...]
    hp = _bn_prelu(hp, gp_ref[...], bep_ref[...], ap_ref[0, 0])
    pred = jnp.dot(hp, wp2_ref[...],
                   preferred_element_type=jnp.float32) + bp2_ref[...]
    sn_ref[...] = _l2n(x2)
    p_ref[...] = _l2n(pred)


def _tc_post(msgsum, h2, dinv_bc, b2, bn_g, bn_b, a2,
             Wp1, bp1, bnp_g, bnp_b, ap, Wp2, bp2):
    return pl.pallas_call(
        _tc_post_body,
        out_shape=[jax.ShapeDtypeStruct((_N, _D), jnp.float32),
                   jax.ShapeDtypeStruct((_N, _D), jnp.float32),
                   jax.ShapeDtypeStruct((_N, _D), jnp.float32)])(
        msgsum, h2, dinv_bc, b2, bn_g, bn_b, a2,
        Wp1, bp1, bnp_g, bnp_b, ap, Wp2, bp2)


_RMINE = 2000


def _tc_mine_body(t_ref, c_ref, o_ref):
    T = t_ref[...]
    C = c_ref[...]
    R = T.shape[0]
    S = T
    d_ = 1
    while d_ < _WC:
        csh = jnp.concatenate(
            [C[:, d_:], jnp.full((R, d_), -2, C.dtype)], axis=1)
        ssh = jnp.concatenate(
            [S[:, d_:], jnp.zeros((R, d_), S.dtype)], axis=1)
        S = S + jnp.where(C == csh, ssh, 0.0)
        d_ *= 2
    cprev = jnp.concatenate(
        [jnp.full((R, 1), -3, C.dtype), C[:, :-1]], axis=1)
    rep = (C != cprev) & (C >= 0)
    cur = jnp.where(rep, S, _NEG)
    iota2 = lax.broadcasted_iota(jnp.int32, (R, _WC), 1)
    li = lax.broadcasted_iota(jnp.int32, (R, _D), 1)
    Cf = C.astype(jnp.float32)
    out = jnp.zeros((R, _D), jnp.float32)
    for i in range(_TOPK - 1):
        mx = jnp.max(cur, axis=1, keepdims=True)
        sel = cur == mx
        firstw = jnp.min(jnp.where(sel, iota2, _WC), axis=1, keepdims=True)
        onehot = iota2 == firstw
        colf = jnp.sum(jnp.where(onehot, Cf, 0.0), axis=1)
        ok = (mx[:, 0] > 0).astype(jnp.float32)
        colf = colf * ok
        out = jnp.where(li == i, colf[:, None], out)
        out = jnp.where(li == 8 + i, ok[:, None], out)
        cur = jnp.where(onehot, _NEG, cur)
    o_ref[...] = out


def _tc_mine(T, C):
    return pl.pallas_call(
        _tc_mine_body,
        grid=(_N // _RMINE,),
        in_specs=[pl.BlockSpec((_RMINE, _WC), lambda i: (i, 0)),
                  pl.BlockSpec((_RMINE, _WC), lambda i: (i, 0))],
        out_specs=pl.BlockSpec((_RMINE, _D), lambda i: (i, 0)),
        out_shape=jax.ShapeDtypeStruct((_N, _D), jnp.float32))(T, C)


def _tc_loss_body(p_ref, t_ref, pd_ref, mine_ref, o_ref):
    p_ = p_ref[...]
    t_ = t_ref[...]
    pd = pd_ref[...]
    mn = mine_ref[...]
    li = lax.broadcasted_iota(jnp.int32, (_N, _D), 1)
    d0 = jnp.sum(p_ * t_, axis=1)
    inv = jnp.sum(2.0 - 2.0 * d0) / _N
    for i in range(_TOPK - 1):
        dots_i = jnp.sum(pd[:_N, i * 16:(i + 1) * 16], axis=1)
        ok_i = jnp.sum(jnp.where(li == 8 + i, mn, 0.0), axis=1)
        cnt = jnp.sum(ok_i)
        inv = inv + jnp.sum(ok_i * (2.0 - 2.0 * dots_i)) / cnt
    inv = inv / _TOPK
    c = lax.dot_general(p_, t_, (((0,), (0,)), ((), ())),
                        preferred_element_type=jnp.float32) / _N
    eye = (lax.broadcasted_iota(jnp.int32, (_D, _D), 0)
           == lax.broadcasted_iota(jnp.int32, (_D, _D), 1))
    on = jnp.sum(jnp.where(eye, (c - 1.0) ** 2, 0.0))
    off = jnp.sum(jnp.where(eye, 0.0, c * c))
    o_ref[...] = jnp.reshape(inv + _LAMBD * (on + off), (1, 1))


def _tc_loss(p, t, pd2d, mined):
    return pl.pallas_call(
        _tc_loss_body,
        out_shape=jax.ShapeDtypeStruct((1, 1), jnp.float32))(
        p, t, pd2d, mined)


_NH = _N // 2


@functools.partial(
    pl.kernel,
    out_type=jax.ShapeDtypeStruct((2, _NC, _NH, _D), jnp.float32),
    scratch_types=[
        pltpu.VMEM((_CCH,), jnp.int32),
        pltpu.VMEM((_CCH,), jnp.int32),
        pltpu.VMEM((_CCH, _D), jnp.float32),
        pltpu.VMEM_SHARED((_NH + 8, _D), jnp.float32),
        pltpu.SemaphoreType.DMA,
    ],
    mesh=_sc_mesh)
def _sc_msg(g_hbm, src_hbm, dst_hbm, out_hbm,
            srcv, dstv, rowsv, accum, sem):
    c = lax.axis_index("c")
    s = lax.axis_index("s")
    wid = s * _NC + c

    for half in range(2):
        lo = half * _NH
        def fillz(i, _):
            rowsv[i // 8, pl.ds((i % 8) * 16, 16)] = (
                jnp.zeros((16,), jnp.float32))
            return 0
        lax.fori_loop(0, 104 * 8, fillz, 0)

        def zslice(k, _):
            pltpu.sync_copy(rowsv.at[pl.ds(0, 104)],
                            accum.at[pl.ds(s * 312 + k * 104, 104)])
            return 0
        lax.fori_loop(0, 3, zslice, 0)
        @pl.when(s == 0)
        def _():
            pltpu.sync_copy(rowsv.at[pl.ds(0, 16)],
                            accum.at[pl.ds(4992, 16)])
        plsc.subcore_barrier()

        def body(it, _):
            b = wid * _EPW + it * _CCH
            pltpu.sync_copy(src_hbm.at[pl.ds(b, _CCH)], srcv)
            pltpu.sync_copy(dst_hbm.at[pl.ds(b, _CCH)], dstv)

            def remap(i, _):
                d = dstv[pl.ds(i * 16, 16)]
                dstv[pl.ds(i * 16, 16)] = jnp.where(
                    (d >= lo) & (d < lo + _NH), d - lo, _NH)
                return 0
            lax.fori_loop(0, _CCH // 16, remap, 0)
            pltpu.async_copy(g_hbm.at[srcv], rowsv, sem).wait()
            pltpu.sync_copy(rowsv, accum.at[dstv], add=True)
            return 0
        lax.fori_loop(0, _NIT, body, 0)
        plsc.subcore_barrier()

        def oslice(k, _):
            pltpu.sync_copy(accum.at[pl.ds(s * 312 + k * 104, 104)],
                            rowsv.at[pl.ds(0, 104)])
            pltpu.sync_copy(
                rowsv.at[pl.ds(0, 104)],
                out_hbm.at[half].at[c].at[pl.ds(s * 312 + k * 104, 104)])
            return 0
        lax.fori_loop(0, 3, oslice, 0)
        @pl.when(s == 0)
        def _():
            pltpu.sync_copy(accum.at[pl.ds(4992, 8)], rowsv.at[pl.ds(0, 8)])
            pltpu.sync_copy(rowsv.at[pl.ds(0, 8)],
                            out_hbm.at[half].at[c].at[pl.ds(4992, 8)])


def kernel(x, y, edge_index, neighbor_index, W1, b1, bn1_g, bn1_b, a1,
           W2, b2, bn2_g, bn2_b, a2, Wp1, bp1, bnp_g, bnp_b, ap, Wp2, bp2):
    N, D = x.shape
    src, dst = edge_index[0], edge_index[1]
    E = src.shape[0]

    degp = _sc_count(dst)
    dinv = jax.lax.rsqrt(degp[:N] + degp[N:] + 1.0)
    dinv_bc = jnp.broadcast_to(dinv[:, None], (N, D))

    def _merge(msgp):
        return jnp.concatenate(
            [msgp[0, 0] + msgp[0, 1], msgp[1, 0] + msgp[1, 1]], axis=0)

    r1 = b1.reshape(1, D)
    g1, h1 = _tc_pre(x, W1, dinv_bc)
    msgsum1 = _merge(_sc_msg(g1, src, dst))
    g2, h2 = _tc_mid(msgsum1, h1, dinv_bc, r1, bn1_g.reshape(1, D),
                     bn1_b.reshape(1, D), a1.reshape(1, 1), W2)
    msgsum2 = _merge(_sc_msg(g2, src, dst))
    student, sn, p = _tc_post(
        msgsum2, h2, dinv_bc, b2.reshape(1, D), bn2_g.reshape(1, D),
        bn2_b.reshape(1, D), a2.reshape(1, 1), Wp1, bp1.reshape(1, -1),
        bnp_g.reshape(1, -1), bnp_b.reshape(1, -1), ap.reshape(1, 1),
        Wp2, bp2.reshape(1, D))
    t = sn

    ns, nd = neighbor_index[0], neighbor_index[1]
    key_s = jnp.sort(ns * N + nd)
    ns_s = key_s // N
    nd_s = key_s - ns_s * N
    sim_s = _sc_edge_sim(ns_s, nd_s, sn, t).reshape(E, 16).sum(axis=1)

    rdegp = _sc_count(ns)
    rdeg = (rdegp[:N] + rdegp[N:]).astype(jnp.int32)
    row_start = jnp.concatenate([jnp.zeros((1,), jnp.int32),
                                 jnp.cumsum(rdeg).astype(jnp.int32)])
    w = jnp.arange(_WC, dtype=jnp.int32)
    flatidx = jnp.minimum(
        row_start[:N, None] + w[None, :], E - 1).reshape(-1)
    t_flat, c_flat = _sc_table(flatidx, sim_s, nd_s)
    validw = w[None, :] < rdeg[:, None]
    T = jnp.where(validw, t_flat.reshape(N, _WC), 0.0)
    C = jnp.where(validw, c_flat.reshape(N, _WC), -1)

    mined = _tc_mine(T, C)
    cols7 = mined[:, :_TOPK - 1].astype(jnp.int32)

    cols8 = jnp.concatenate(
        [cols7, jnp.zeros((N, 1), jnp.int32)], axis=1)
    cols_pad = jnp.concatenate(
        [cols8, jnp.zeros((_NPAIR // 8 - N, 8), jnp.int32)], axis=0)
    rowidx = jnp.minimum(jnp.arange(_NPAIR, dtype=jnp.int32) // 8, N - 1)
    pd2d = _sc_pair_dot(
        rowidx, cols_pad.reshape(-1), p, t).reshape(_NPAIR // 8, 8 * 16)
    loss = _tc_loss(p, t, pd2d, mined)[0, 0]
    return (student, loss)


# speedup vs baseline: 20.7227x; 1.0083x over previous
import functools

import jax
import jax.numpy as jnp
from jax import lax
from jax.experimental import pallas as pl
from jax.experimental.pallas import tpu as pltpu
from jax.experimental.pallas import tpu_sc as plsc

_TOPK = 8
_LAMBD = 0.001
_WC = 256

_N, _E, _D = 10000, 320000, 128
_NC, _NS, _NW = 2, 16, 32
_EPW = _E // _NW
_CCH = 400
_NIT = _EPW // _CCH

_sc_mesh = plsc.VectorSubcoreMesh(core_axis_name="c", subcore_axis_name="s")


@functools.partial(
    pl.kernel,
    out_type=jax.ShapeDtypeStruct((_NC * _N,), jnp.float32),
    scratch_types=[
        pltpu.VMEM((_CCH,), jnp.int32),
        pltpu.VMEM((_CCH,), jnp.float32),
        pltpu.VMEM((624,), jnp.float32),
        pltpu.VMEM_SHARED((_N,), jnp.float32),
    ],
    mesh=_sc_mesh)
def _sc_count(idx_hbm, out_hbm, idxv, onesv, tbuf, accum):
    c = lax.axis_index("c")
    s = lax.axis_index("s")
    wid = s * _NC + c

    def fill(i, _):
        onesv[pl.ds(i * 16, 16)] = jnp.full((16,), 1.0, jnp.float32)
        return 0
    lax.fori_loop(0, _CCH // 16, fill, 0)

    def fillz(i, _):
        tbuf[pl.ds(i * 16, 16)] = jnp.zeros((16,), jnp.float32)
        return 0
    lax.fori_loop(0, 624 // 16, fillz, 0)

    pltpu.sync_copy(tbuf, accum.at[pl.ds(s * 624, 624)])
    @pl.when(s == 0)
    def _():
        pltpu.sync_copy(tbuf.at[pl.ds(0, 16)], accum.at[pl.ds(9984, 16)])
    plsc.subcore_barrier()

    def body(it, _):
        b = wid * _EPW + it * _CCH
        pltpu.sync_copy(idx_hbm.at[pl.ds(b, _CCH)], idxv)
        pltpu.sync_copy(onesv, accum.at[idxv], add=True)
        return 0
    lax.fori_loop(0, _NIT, body, 0)
    plsc.subcore_barrier()

    off = c * _N
    pltpu.sync_copy(accum.at[pl.ds(s * 624, 624)], tbuf)
    pltpu.sync_copy(tbuf, out_hbm.at[pl.ds(off + s * 624, 624)])
    @pl.when(s == 0)
    def _():
        pltpu.sync_copy(accum.at[pl.ds(9984, 16)], tbuf.at[pl.ds(0, 16)])
        pltpu.sync_copy(tbuf.at[pl.ds(0, 16)], out_hbm.at[pl.ds(off + 9984, 16)])


_NPAIR = 81920
_RPW = 313
def _make_pairdot(total, cch):
    epw = total // _NW
    nit = epw // cch

    @functools.partial(
        pl.kernel,
        out_type=jax.ShapeDtypeStruct((total * 16,), jnp.float32),
        scratch_types=[
            pltpu.VMEM((cch,), jnp.int32),
            pltpu.VMEM((cch,), jnp.int32),
            pltpu.VMEM((cch, _D), jnp.float32),
            pltpu.VMEM((cch, _D), jnp.float32),
            pltpu.VMEM((cch * 16,), jnp.float32),
            pltpu.SemaphoreType.DMA,
            pltpu.SemaphoreType.DMA,
        ],
        mesh=_sc_mesh)
    def dotk(ns_hbm, nd_hbm, a_hbm, b_hbm, sim_out,
             nsv, ndv, ra, rb, simv, sema, semb):
        c = lax.axis_index("c")
        s = lax.axis_index("s")
        wid = s * _NC + c

        def body(it, _):
            bofs = wid * epw + it * cch
            pltpu.sync_copy(ns_hbm.at[pl.ds(bofs, cch)], nsv)
            pltpu.sync_copy(nd_hbm.at[pl.ds(bofs, cch)], ndv)
            cpa = pltpu.async_copy(a_hbm.at[nsv], ra, sema)
            cpb = pltpu.async_copy(b_hbm.at[ndv], rb, semb)
            cpa.wait()
            cpb.wait()

            def dote(e, _):
                acc = ra[e, pl.ds(0, 16)] * rb[e, pl.ds(0, 16)]
                for j in range(1, _D // 16):
                    acc = acc + (ra[e, pl.ds(j * 16, 16)]
                                 * rb[e, pl.ds(j * 16, 16)])
                simv[pl.ds(e * 16, 16)] = acc
                return 0
            lax.fori_loop(0, cch, dote, 0)

            pltpu.sync_copy(simv, sim_out.at[pl.ds(bofs * 16, cch * 16)])
            return 0
        lax.fori_loop(0, nit, body, 0)
    return dotk


def _make_pairdot_db(total, cch):
    epw = total // _NW
    nit = epw // cch

    @functools.partial(
        pl.kernel,
        out_type=jax.ShapeDtypeStruct((total * 16,), jnp.float32),
        scratch_types=[
            pltpu.VMEM((cch,), jnp.int32),
            pltpu.VMEM((cch,), jnp.int32),
            pltpu.VMEM((cch,), jnp.int32),
            pltpu.VMEM((cch,), jnp.int32),
            pltpu.VMEM((cch, _D), jnp.float32),
            pltpu.VMEM((cch, _D), jnp.float32),
            pltpu.VMEM((cch, _D), jnp.float32),
            pltpu.VMEM((cch, _D), jnp.float32),
            pltpu.VMEM((cch * 16,), jnp.float32),
            pltpu.SemaphoreType.DMA,
            pltpu.SemaphoreType.DMA,
            pltpu.SemaphoreType.DMA,
            pltpu.SemaphoreType.DMA,
        ],
        mesh=_sc_mesh)
    def dotk(ns_hbm, nd_hbm, a_hbm, b_hbm, sim_out,
             nsv0, ndv0, nsv1, ndv1, ra0, rb0, ra1, rb1, simv,
             sa0, sb0, sa1, sb1):
        c = lax.axis_index("c")
        s = lax.axis_index("s")
        wid = s * _NC + c
        bufs = ((nsv0, ndv0, ra0, rb0, sa0, sb0),
                (nsv1, ndv1, ra1, rb1, sa1, sb1))

        def launch(i, bi):
            nsv, ndv, ra, rb, sa, sb = bufs[bi]
            b = wid * epw + i * cch
            pltpu.sync_copy(ns_hbm.at[pl.ds(b, cch)], nsv)
            pltpu.sync_copy(nd_hbm.at[pl.ds(b, cch)], ndv)
            pltpu.async_copy(a_hbm.at[nsv], ra, sa)
            pltpu.async_copy(b_hbm.at[ndv], rb, sb)

        def finish(i, bi):
            nsv, ndv, ra, rb, sa, sb = bufs[bi]
            pltpu.make_async_copy(a_hbm.at[nsv], ra, sa).wait()
            pltpu.make_async_copy(b_hbm.at[ndv], rb, sb).wait()

            def dote(e, _):
                acc = ra[e, pl.ds(0, 16)] * rb[e, pl.ds(0, 16)]
                for j in range(1, _D // 16):
                    acc = acc + (ra[e, pl.ds(j * 16, 16)]
                                 * rb[e, pl.ds(j * 16, 16)])
                simv[pl.ds(e * 16, 16)] = acc
                return 0
            lax.fori_loop(0, cch, dote, 0)
            b = wid * epw + i * cch
            pltpu.sync_copy(simv, sim_out.at[pl.ds(b * 16, cch * 16)])

        launch(0, 0)

        def body(k, _):
            i0 = k * 2
            launch(i0 + 1, 1)
            finish(i0, 0)

            @pl.when(i0 + 2 < nit)
            def _():
                launch(i0 + 2, 0)
            finish(i0 + 1, 1)
            return 0
        lax.fori_loop(0, nit // 2, body, 0)
    return dotk


_sc_edge_sim = _make_pairdot_db(_E, 200)
_sc_pair_dot = _make_pairdot_db(_NPAIR, 160)


_TPW = _N * _WC // _NW


@functools.partial(
    pl.kernel,
    out_type=[jax.ShapeDtypeStruct((_N * _WC,), jnp.float32),
              jax.ShapeDtypeStruct((_N * _WC,), jnp.int32)],
    scratch_types=[
        pltpu.VMEM((2000,), jnp.int32),
        pltpu.VMEM((2000,), jnp.float32),
        pltpu.VMEM((2000,), jnp.int32),
        pltpu.SemaphoreType.DMA,
        pltpu.SemaphoreType.DMA,
    ],
    mesh=_sc_mesh)
def _sc_table(flat_hbm, sim_hbm, nd_hbm, t_out, c_out,
              idxb, tvb, cvb, sema, semb):
    c = lax.axis_index("c")
    s = lax.axis_index("s")
    wid = s * _NC + c

    def body(it, _):
        b = wid * _TPW + it * 2000
        pltpu.sync_copy(flat_hbm.at[pl.ds(b, 2000)], idxb)
        pltpu.async_copy(sim_hbm.at[idxb], tvb, sema).wait()
        pltpu.async_copy(nd_hbm.at[idxb], cvb, semb).wait()
        pltpu.sync_copy(tvb, t_out.at[pl.ds(b, 2000)])
        pltpu.sync_copy(cvb, c_out.at[pl.ds(b, 2000)])
        return 0
    lax.fori_loop(0, _TPW // 2000, body, 0)


_NEG = -1e30


def _bn_prelu(out, g, beta, a):
    m = jnp.mean(out, axis=0)
    d = out - m
    v = jnp.mean(d * d, axis=0)
    y = d / jnp.sqrt(v + 1e-5) * g + beta
    return jnp.where(y >= 0, y, a * y)


def _tc_pre_body(x_ref, w_ref, dv_ref, g_ref, h_ref):
    h = jnp.dot(x_ref[...], w_ref[...], preferred_element_type=jnp.float32)
    h_ref[...] = h
    g_ref[...] = h * dv_ref[...]


def _tc_pre(x, W1, dinv_bc):
    return pl.pallas_call(
        _tc_pre_body,
        out_shape=[jax.ShapeDtypeStruct((_N, _D), jnp.float32),
                   jax.ShapeDtypeStruct((_N, _D), jnp.float32)])(
        x, W1, dinv_bc)


def _tc_mid_body(msg_ref, h_ref, dv_ref, b_ref, g_ref, be_ref, a_ref, w_ref,
                 g2_ref, h2_ref):
    dv = dv_ref[...]
    out = dv * msg_ref[...] + dv * dv * h_ref[...] + b_ref[...]
    x1 = _bn_prelu(out, g_ref[...], be_ref[...], a_ref[0, 0])
    h2 = jnp.dot(x1, w_ref[...], preferred_element_type=jnp.float32)
    h2_ref[...] = h2
    g2_ref[...] = h2 * dv


def _tc_mid(msgsum, h1, dinv_bc, b1, bn_g, bn_b, a1, W2):
    return pl.pallas_call(
        _tc_mid_body,
        out_shape=[jax.ShapeDtypeStruct((_N, _D), jnp.float32),
                   jax.ShapeDtypeStruct((_N, _D), jnp.float32)])(
        msgsum, h1, dinv_bc, b1, bn_g, bn_b, a1, W2)


def _l2n(v):
    n = jnp.sqrt(jnp.sum(v * v, axis=1, keepdims=True))
    return v / jnp.maximum(n, 1e-12)


def _tc_post_body(msg_ref, h_ref, dv_ref, b_ref, g_ref, be_ref, a_ref,
                  wp1_ref, bp1_ref, gp_ref, bep_ref, ap_ref, wp2_ref,
                  bp2_ref, stu_ref, sn_ref, p_ref):
    dv = dv_ref[...]
    out = dv * msg_ref[...] + dv * dv * h_ref[...] + b_ref[...]
    x2 = _bn_prelu(out, g_ref[...], be_ref[...], a_ref[0, 0])
    stu_ref[...] = x2
    hp = jnp.dot(x2, wp1_ref[...],
                 preferred_element_type=jnp.float32) + bp1_ref[...]
    hp = _bn_prelu(hp, gp_ref[...], bep_ref[...], ap_ref[0, 0])
    pred = jnp.dot(hp, wp2_ref[...],
                   preferred_element_type=jnp.float32) + bp2_ref[...]
    sn_ref[...] = _l2n(x2)
    p_ref[...] = _l2n(pred)


def _tc_post(msgsum, h2, dinv_bc, b2, bn_g, bn_b, a2,
             Wp1, bp1, bnp_g, bnp_b, ap, Wp2, bp2):
    return pl.pallas_call(
        _tc_post_body,
        out_shape=[jax.ShapeDtypeStruct((_N, _D), jnp.float32),
                   jax.ShapeDtypeStruct((_N, _D), jnp.float32),
                   jax.ShapeDtypeStruct((_N, _D), jnp.float32)])(
        msgsum, h2, dinv_bc, b2, bn_g, bn_b, a2,
        Wp1, bp1, bnp_g, bnp_b, ap, Wp2, bp2)


_RMINE = 2000


def _tc_mine_body(t_ref, c_ref, o_ref):
    T = t_ref[...]
    C = c_ref[...]
    R = T.shape[0]
    S = T
    d_ = 1
    while d_ < _WC:
        csh = jnp.concatenate(
            [C[:, d_:], jnp.full((R, d_), -2, C.dtype)], axis=1)
        ssh = jnp.concatenate(
            [S[:, d_:], jnp.zeros((R, d_), S.dtype)], axis=1)
        S = S + jnp.where(C == csh, ssh, 0.0)
        d_ *= 2
    cprev = jnp.concatenate(
        [jnp.full((R, 1), -3, C.dtype), C[:, :-1]], axis=1)
    rep = (C != cprev) & (C >= 0)
    cur = jnp.where(rep, S, _NEG)
    iota2 = lax.broadcasted_iota(jnp.int32, (R, _WC), 1)
    li = lax.broadcasted_iota(jnp.int32, (R, _D), 1)
    Cf = C.astype(jnp.float32)
    out = jnp.zeros((R, _D), jnp.float32)
    for i in range(_TOPK - 1):
        mx = jnp.max(cur, axis=1, keepdims=True)
        sel = cur == mx
        firstw = jnp.min(jnp.where(sel, iota2, _WC), axis=1, keepdims=True)
        onehot = iota2 == firstw
        colf = jnp.sum(jnp.where(onehot, Cf, 0.0), axis=1)
        ok = (mx[:, 0] > 0).astype(jnp.float32)
        colf = colf * ok
        out = jnp.where(li == i, colf[:, None], out)
        out = jnp.where(li == 8 + i, ok[:, None], out)
        cur = jnp.where(onehot, _NEG, cur)
    o_ref[...] = out


def _tc_mine(T, C):
    return pl.pallas_call(
        _tc_mine_body,
        grid=(_N // _RMINE,),
        in_specs=[pl.BlockSpec((_RMINE, _WC), lambda i: (i, 0)),
                  pl.BlockSpec((_RMINE, _WC), lambda i: (i, 0))],
        out_specs=pl.BlockSpec((_RMINE, _D), lambda i: (i, 0)),
        out_shape=jax.ShapeDtypeStruct((_N, _D), jnp.float32))(T, C)


def _tc_loss_body(p_ref, t_ref, pd_ref, mine_ref, o_ref):
    p_ = p_ref[...]
    t_ = t_ref[...]
    pd = pd_ref[...]
    mn = mine_ref[...]
    li = lax.broadcasted_iota(jnp.int32, (_N, _D), 1)
    d0 = jnp.sum(p_ * t_, axis=1)
    inv = jnp.sum(2.0 - 2.0 * d0) / _N
    for i in range(_TOPK - 1):
        dots_i = jnp.sum(pd[:_N, i * 16:(i + 1) * 16], axis=1)
        ok_i = jnp.sum(jnp.where(li == 8 + i, mn, 0.0), axis=1)
        cnt = jnp.sum(ok_i)
        inv = inv + jnp.sum(ok_i * (2.0 - 2.0 * dots_i)) / cnt
    inv = inv / _TOPK
    c = lax.dot_general(p_, t_, (((0,), (0,)), ((), ())),
                        preferred_element_type=jnp.float32) / _N
    eye = (lax.broadcasted_iota(jnp.int32, (_D, _D), 0)
           == lax.broadcasted_iota(jnp.int32, (_D, _D), 1))
    on = jnp.sum(jnp.where(eye, (c - 1.0) ** 2, 0.0))
    off = jnp.sum(jnp.where(eye, 0.0, c * c))
    o_ref[...] = jnp.reshape(inv + _LAMBD * (on + off), (1, 1))


def _tc_loss(p, t, pd2d, mined):
    return pl.pallas_call(
        _tc_loss_body,
        out_shape=jax.ShapeDtypeStruct((1, 1), jnp.float32))(
        p, t, pd2d, mined)


_NH = _N // 2


@functools.partial(
    pl.kernel,
    out_type=jax.ShapeDtypeStruct((2, _NC, _NH, _D), jnp.float32),
    scratch_types=[
        pltpu.VMEM((_CCH,), jnp.int32),
        pltpu.VMEM((_CCH,), jnp.int32),
        pltpu.VMEM((_CCH, _D), jnp.float32),
        pltpu.VMEM_SHARED((_NH + 8, _D), jnp.float32),
        pltpu.SemaphoreType.DMA,
    ],
    mesh=_sc_mesh)
def _sc_msg(g_hbm, src_hbm, dst_hbm, out_hbm,
            srcv, dstv, rowsv, accum, sem):
    c = lax.axis_index("c")
    s = lax.axis_index("s")
    wid = s * _NC + c

    for half in range(2):
        lo = half * _NH
        def fillz(i, _):
            rowsv[i // 8, pl.ds((i % 8) * 16, 16)] = (
                jnp.zeros((16,), jnp.float32))
            return 0
        lax.fori_loop(0, 104 * 8, fillz, 0)

        def zslice(k, _):
            pltpu.sync_copy(rowsv.at[pl.ds(0, 104)],
                            accum.at[pl.ds(s * 312 + k * 104, 104)])
            return 0
        lax.fori_loop(0, 3, zslice, 0)
        @pl.when(s == 0)
        def _():
            pltpu.sync_copy(rowsv.at[pl.ds(0, 16)],
                            accum.at[pl.ds(4992, 16)])
        plsc.subcore_barrier()

        def body(it, _):
            b = wid * _EPW + it * _CCH
            pltpu.sync_copy(src_hbm.at[pl.ds(b, _CCH)], srcv)
            pltpu.sync_copy(dst_hbm.at[pl.ds(b, _CCH)], dstv)

            def remap(i, _):
                d = dstv[pl.ds(i * 16, 16)]
                dstv[pl.ds(i * 16, 16)] = jnp.where(
                    (d >= lo) & (d < lo + _NH), d - lo, _NH)
                return 0
            lax.fori_loop(0, _CCH // 16, remap, 0)
            pltpu.async_copy(g_hbm.at[srcv], rowsv, sem).wait()
            pltpu.sync_copy(rowsv, accum.at[dstv], add=True)
            return 0
        lax.fori_loop(0, _NIT, body, 0)
        plsc.subcore_barrier()

        def oslice(k, _):
            pltpu.sync_copy(accum.at[pl.ds(s * 312 + k * 104, 104)],
                            rowsv.at[pl.ds(0, 104)])
            pltpu.sync_copy(
                rowsv.at[pl.ds(0, 104)],
                out_hbm.at[half].at[c].at[pl.ds(s * 312 + k * 104, 104)])
            return 0
        lax.fori_loop(0, 3, oslice, 0)
        @pl.when(s == 0)
        def _():
            pltpu.sync_copy(accum.at[pl.ds(4992, 8)], rowsv.at[pl.ds(0, 8)])
            pltpu.sync_copy(rowsv.at[pl.ds(0, 8)],
                            out_hbm.at[half].at[c].at[pl.ds(4992, 8)])


def kernel(x, y, edge_index, neighbor_index, W1, b1, bn1_g, bn1_b, a1,
           W2, b2, bn2_g, bn2_b, a2, Wp1, bp1, bnp_g, bnp_b, ap, Wp2, bp2):
    N, D = x.shape
    src, dst = edge_index[0], edge_index[1]
    E = src.shape[0]

    degp = _sc_count(dst)
    dinv = jax.lax.rsqrt(degp[:N] + degp[N:] + 1.0)
    dinv_bc = jnp.broadcast_to(dinv[:, None], (N, D))

    def _merge(msgp):
        return jnp.concatenate(
            [msgp[0, 0] + msgp[0, 1], msgp[1, 0] + msgp[1, 1]], axis=0)

    r1 = b1.reshape(1, D)
    g1, h1 = _tc_pre(x, W1, dinv_bc)
    msgsum1 = _merge(_sc_msg(g1, src, dst))
    g2, h2 = _tc_mid(msgsum1, h1, dinv_bc, r1, bn1_g.reshape(1, D),
                     bn1_b.reshape(1, D), a1.reshape(1, 1), W2)
    msgsum2 = _merge(_sc_msg(g2, src, dst))
    student, sn, p = _tc_post(
        msgsum2, h2, dinv_bc, b2.reshape(1, D), bn2_g.reshape(1, D),
        bn2_b.reshape(1, D), a2.reshape(1, 1), Wp1, bp1.reshape(1, -1),
        bnp_g.reshape(1, -1), bnp_b.reshape(1, -1), ap.reshape(1, 1),
        Wp2, bp2.reshape(1, D))
    t = sn

    ns, nd = neighbor_index[0], neighbor_index[1]
    key_s = jnp.sort(ns * N + nd)
    ns_s = key_s // N
    nd_s = key_s - ns_s * N
    sim_s = _sc_edge_sim(ns_s, nd_s, sn, t).reshape(E, 16).sum(axis=1)

    rdegp = _sc_count(ns)
    rdeg = (rdegp[:N] + rdegp[N:]).astype(jnp.int32)
    row_start = jnp.concatenate([jnp.zeros((1,), jnp.int32),
                                 jnp.cumsum(rdeg).astype(jnp.int32)])
    w = jnp.arange(_WC, dtype=jnp.int32)
    flatidx = jnp.minimum(
        row_start[:N, None] + w[None, :], E - 1).reshape(-1)
    t_flat, c_flat = _sc_table(flatidx, sim_s, nd_s)
    validw = w[None, :] < rdeg[:, None]
    T = jnp.where(validw, t_flat.reshape(N, _WC), 0.0)
    C = jnp.where(validw, c_flat.reshape(N, _WC), -1)

    mined = _tc_mine(T, C)
    cols7 = mined[:, :_TOPK - 1].astype(jnp.int32)

    cols8 = jnp.concatenate(
        [cols7, jnp.zeros((N, 1), jnp.int32)], axis=1)
    cols_pad = jnp.concatenate(
        [cols8, jnp.zeros((_NPAIR // 8 - N, 8), jnp.int32)], axis=0)
    rowidx = jnp.minimum(jnp.arange(_NPAIR, dtype=jnp.int32) // 8, N - 1)
    pd2d = _sc_pair_dot(
        rowidx, cols_pad.reshape(-1), p, t).reshape(_NPAIR // 8, 8 * 16)
    loss = _tc_loss(p, t, pd2d, mined)[0, 0]
    return (student, loss)
